# baseline (speedup 1.0000x reference)
import functools

import jax
import jax.numpy as jnp
from jax import lax
from jax.experimental import pallas as pl
from jax.experimental.pallas import tpu as pltpu
from jax.experimental.pallas import tpu_sc as plsc

_B = 256
_D = 4 * 64 * 64
_NCH = 32
_RW = _B // _NCH
_NBUF = 4


def _sc_gather_body(a_hbm, b_hbm, t_hbm, a_out, b_out, idx_v, rows_a, rows_b, sem):
    info = plsc.get_sparse_core_info()
    nc = info.num_cores
    wid = lax.axis_index("s") * nc + lax.axis_index("c")
    nw = nc * info.num_subcores
    bw = _B // nw
    base = wid * bw
    pltpu.sync_copy(t_hbm.at[pl.ds(base, bw)], idx_v)
    cp_a = pltpu.async_copy(a_hbm.at[idx_v], rows_a, sem)
    cp_b = pltpu.async_copy(b_hbm.at[idx_v], rows_b, sem)
    cp_a.wait()
    cp_b.wait()
    pltpu.sync_copy(rows_a, a_out.at[pl.ds(base, bw)])
    pltpu.sync_copy(rows_b, b_out.at[pl.ds(base, bw)])


def _sc_gather(a_tbl, b_tbl, t):
    info = plsc.get_sparse_core_info()
    nw = info.num_cores * info.num_subcores
    bw = _B // nw
    mesh = plsc.VectorSubcoreMesh(core_axis_name="c", subcore_axis_name="s")
    f = functools.partial(
        pl.kernel,
        mesh=mesh,
        out_type=(
            jax.ShapeDtypeStruct((_B,), jnp.float32),
            jax.ShapeDtypeStruct((_B,), jnp.float32),
        ),
        scratch_types=[
            pltpu.VMEM((bw,), jnp.int32),
            pltpu.VMEM((bw,), jnp.float32),
            pltpu.VMEM((bw,), jnp.float32),
            pltpu.SemaphoreType.DMA,
        ],
    )(_sc_gather_body)
    return f(a_tbl, b_tbl, t)


def _fma_body(a_ref, b_ref, x_hbm, n_hbm, o_hbm, xv, nv, ov, sx, sn, so):
    def start_in(c):
        s = c % _NBUF
        pltpu.make_async_copy(x_hbm.at[pl.ds(c * _RW, _RW)], xv.at[s], sx.at[s]).start()
        pltpu.make_async_copy(n_hbm.at[pl.ds(c * _RW, _RW)], nv.at[s], sn.at[s]).start()

    for c in range(_NBUF):
        start_in(c)
    for c in range(_NCH):
        s = c % _NBUF
        pltpu.make_async_copy(x_hbm.at[pl.ds(c * _RW, _RW)], xv.at[s], sx.at[s]).wait()
        pltpu.make_async_copy(n_hbm.at[pl.ds(c * _RW, _RW)], nv.at[s], sn.at[s]).wait()
        if c >= _NBUF:
            pltpu.make_async_copy(
                ov.at[s], o_hbm.at[pl.ds((c - _NBUF) * _RW, _RW)], so.at[s]
            ).wait()
        a = a_ref[pl.ds(c * _RW, _RW), :]
        b = b_ref[pl.ds(c * _RW, _RW), :]
        ov[s] = a * xv[s] + b * nv[s]
        pltpu.make_async_copy(ov.at[s], o_hbm.at[pl.ds(c * _RW, _RW)], so.at[s]).start()
        if c + _NBUF < _NCH:
            start_in(c + _NBUF)
    for c in range(_NCH - _NBUF, _NCH):
        s = c % _NBUF
        pltpu.make_async_copy(ov.at[s], o_hbm.at[pl.ds(c * _RW, _RW)], so.at[s]).wait()


def _fma(a_g, b_g, x2, n2):
    return pl.pallas_call(
        _fma_body,
        in_specs=[
            pl.BlockSpec(memory_space=pltpu.VMEM),
            pl.BlockSpec(memory_space=pltpu.VMEM),
            pl.BlockSpec(memory_space=pl.ANY),
            pl.BlockSpec(memory_space=pl.ANY),
        ],
        out_specs=pl.BlockSpec(memory_space=pl.ANY),
        out_shape=jax.ShapeDtypeStruct((_B, _D), jnp.float32),
        scratch_shapes=[
            pltpu.VMEM((_NBUF, _RW, _D), jnp.float32),
            pltpu.VMEM((_NBUF, _RW, _D), jnp.float32),
            pltpu.VMEM((_NBUF, _RW, _D), jnp.float32),
            pltpu.SemaphoreType.DMA((_NBUF,)),
            pltpu.SemaphoreType.DMA((_NBUF,)),
            pltpu.SemaphoreType.DMA((_NBUF,)),
        ],
    )(a_g, b_g, x2, n2)


def kernel(x_start, t, noise, sqrt_alphas_cumprod, sqrt_one_minus_alphas_cumprod):
    a_g, b_g = _sc_gather(
        sqrt_alphas_cumprod.astype(jnp.float32),
        sqrt_one_minus_alphas_cumprod.astype(jnp.float32),
        t.astype(jnp.int32),
    )
    x2 = x_start.reshape(_B, _D)
    n2 = noise.reshape(_B, _D)
    out = _fma(a_g.reshape(_B, 1), b_g.reshape(_B, 1), x2, n2)
    return out.reshape(x_start.shape)
